# baseline (speedup 1.0000x reference)
import functools

import jax
import jax.numpy as jnp
from jax.experimental import pallas as pl
from jax.experimental.pallas import tpu as pltpu

_BN_EPS = 1e-5
_VMEM_LIMIT = 64 * 1024 * 1024
_IPB = 4


def _build_patches(x3, xc_ref, H, W, Cin):
    R = H * W
    for kh in range(3):
        for kw in range(3):
            t = kh * 3 + kw
            v = x3[kh:kh + H, kw:kw + W, :].reshape(R, Cin)
            xc_ref[:, t * Cin:(t + 1) * Cin] = v


def _stats_kernel(H, W, Cin, x_ref, w_ref, stats_ref, xc_ref):
    s_acc = None
    for j in range(_IPB):
        _build_patches(x_ref[j], xc_ref, H, W, Cin)
        y = jnp.dot(xc_ref[...], w_ref[...], preferred_element_type=jnp.float32)
        s = jnp.sum(y, axis=0)
        ss = jnp.sum(y * y, axis=0)
        s_acc = (s, ss) if s_acc is None else (s_acc[0] + s, s_acc[1] + ss)
    stats_ref[0, 0, :] = s_acc[0]
    stats_ref[0, 1, :] = s_acc[1]


def _out_kernel(H, W, Cin, x_ref, w_ref, scale_ref, shift_ref, o_ref, xc_ref):
    for j in range(_IPB):
        _build_patches(x_ref[j], xc_ref, H, W, Cin)
        yt = jax.lax.dot_general(
            w_ref[...], xc_ref[...],
            dimension_numbers=(((0,), (1,)), ((), ())),
            preferred_element_type=jnp.float32)
        o_ref[j] = jnp.maximum(yt * scale_ref[...] + shift_ref[...], 0.0)


def kernel(x_nchw, w_oihw, gamma, beta):
    N, Cin, H, W = x_nchw.shape
    Cout = w_oihw.shape[0]
    K = 9 * Cin
    R = H * W
    G = N // _IPB

    x_nhwc = jnp.transpose(x_nchw, (0, 2, 3, 1)).astype(jnp.bfloat16)
    xp = jnp.pad(x_nhwc, ((0, 0), (1, 1), (1, 1), (0, 0)))
    w_mat = jnp.transpose(w_oihw, (2, 3, 1, 0)).reshape(K, Cout).astype(jnp.bfloat16)

    params = pltpu.CompilerParams(
        dimension_semantics=("arbitrary",),
        vmem_limit_bytes=_VMEM_LIMIT)

    stats = pl.pallas_call(
        functools.partial(_stats_kernel, H, W, Cin),
        out_shape=jax.ShapeDtypeStruct((G, 2, Cout), jnp.float32),
        grid=(G,),
        in_specs=[
            pl.BlockSpec((_IPB, H + 2, W + 2, Cin), lambda i: (i, 0, 0, 0)),
            pl.BlockSpec((K, Cout), lambda i: (0, 0)),
        ],
        out_specs=pl.BlockSpec((1, 2, Cout), lambda i: (i, 0, 0)),
        scratch_shapes=[pltpu.VMEM((R, K), jnp.bfloat16)],
        compiler_params=params,
    )(xp, w_mat)

    tot = jnp.sum(stats, axis=0)
    cnt = jnp.float32(N * R)
    mean = tot[0] / cnt
    var = tot[1] / cnt - mean * mean
    inv_std = jax.lax.rsqrt(var + _BN_EPS)
    scale = (gamma.astype(jnp.float32) * inv_std).reshape(Cout, 1)
    shift = (beta.astype(jnp.float32) - mean * gamma.astype(jnp.float32)
             * inv_std).reshape(Cout, 1)

    out_flat = pl.pallas_call(
        functools.partial(_out_kernel, H, W, Cin),
        out_shape=jax.ShapeDtypeStruct((N, Cout, R), jnp.float32),
        grid=(G,),
        in_specs=[
            pl.BlockSpec((_IPB, H + 2, W + 2, Cin), lambda i: (i, 0, 0, 0)),
            pl.BlockSpec((K, Cout), lambda i: (0, 0)),
            pl.BlockSpec((Cout, 1), lambda i: (0, 0)),
            pl.BlockSpec((Cout, 1), lambda i: (0, 0)),
        ],
        out_specs=pl.BlockSpec((_IPB, Cout, R), lambda i: (i, 0, 0)),
        scratch_shapes=[pltpu.VMEM((R, K), jnp.bfloat16)],
        compiler_params=params,
    )(xp, w_mat, scale, shift)

    return out_flat.reshape(N, Cout, H, W)
